# baseline (speedup 1.0000x reference)
import functools

import jax
import jax.numpy as jnp
from jax import lax
from jax.experimental import pallas as pl
from jax.experimental.pallas import tpu as pltpu
from jax.experimental.pallas import tpu_sc as plsc

B = 128
V = 100000
K = 256

NC = 2
NS = 16
L = 16
NW = NC * NS
ROWS_PER_W = B // NW


def kernel(input_ids, scores, allowed_idx):
    del input_ids
    scores_flat = scores.reshape(-1)

    mesh = plsc.VectorSubcoreMesh(
        core_axis_name="c", subcore_axis_name="s", num_cores=NC,
        num_subcores=NS)

    @functools.partial(
        pl.kernel,
        out_type=jax.ShapeDtypeStruct((B, V), jnp.float32),
        mesh=mesh,
        compiler_params=pltpu.CompilerParams(needs_layout_passes=False),
        scratch_types=[
            pltpu.VMEM((V,), jnp.float32),
            pltpu.VMEM((K,), jnp.int32),
            pltpu.VMEM((2, 128), jnp.int32),
            pltpu.VMEM((2, 128), jnp.float32),
            pltpu.SemaphoreType.DMA,
        ],
    )
    def sc_kernel(scores_hbm, idx_hbm, out_hbm, row_v, idx_v, fidx_v,
                  vals_v, sem):
        wid = lax.axis_index("s") * NC + lax.axis_index("c")
        neg_inf = jnp.full((L,), -jnp.inf, jnp.float32)

        def fill(i, carry):
            row_v[pl.ds(i * L, L)] = neg_inf
            return carry

        lax.fori_loop(0, V // L, fill, 0)

        for r in range(ROWS_PER_W):
            b = wid * ROWS_PER_W + r
            pltpu.sync_copy(idx_hbm.at[b], idx_v)
            for j in range(K // L):
                iv = idx_v[pl.ds(j * L, L)]
                fidx_v[j // 8, pl.ds((j % 8) * L, L)] = iv + b * V
            for h in range(2):
                pltpu.async_copy(
                    scores_hbm.at[fidx_v.at[h]], vals_v.at[h], sem).wait()
            for j in range(K // L):
                iv = idx_v[pl.ds(j * L, L)]
                vv = vals_v[j // 8, pl.ds((j % 8) * L, L)]
                plsc.store_scatter(row_v, [iv], vv)
            pltpu.sync_copy(row_v, out_hbm.at[b])
            for j in range(K // L):
                iv = idx_v[pl.ds(j * L, L)]
                plsc.store_scatter(row_v, [iv], neg_inf)

    return sc_kernel(scores_flat, allowed_idx)


# speedup vs baseline: 1.2824x; 1.2824x over previous
import jax
import jax.numpy as jnp
from jax import lax
from jax.experimental import pallas as pl
from jax.experimental.pallas import tpu as pltpu
from jax.experimental.pallas import tpu_sc as plsc

B = 128
V = 100000
K = 256

NC = 2
NS = 16
L = 16
ROWS = 8
CHW = 2048
NK = 24
NG = NK // 2
TAIL_LO = 2 * NK * CHW
TAIL_N = V - TAIL_LO
TAIL_A = (TAIL_N // 128) * 128
TAIL_B = TAIL_N - TAIL_A
TB_LO = TAIL_LO + TAIL_A


def kernel(input_ids, scores, allowed_idx):
    del input_ids

    mesh = plsc.VectorSubcoreMesh(
        core_axis_name="c", subcore_axis_name="s", num_cores=NC,
        num_subcores=NS)

    @pl.kernel(
        out_type=jax.ShapeDtypeStruct((B, V), jnp.float32),
        mesh=mesh,
        compiler_params=pltpu.CompilerParams(needs_layout_passes=False),
        scratch_types=[
            pltpu.VMEM((2, ROWS, CHW), jnp.float32),
            pltpu.VMEM((2, ROWS, CHW), jnp.float32),
            pltpu.VMEM((ROWS, K), jnp.int32),
            pltpu.VMEM((ROWS, TAIL_B), jnp.float32),
            pltpu.VMEM((ROWS, TAIL_B), jnp.float32),
            pltpu.SemaphoreType.DMA,
            pltpu.SemaphoreType.DMA,
            pltpu.SemaphoreType.DMA,
            pltpu.SemaphoreType.DMA,
        ],
    )
    def sc_kernel(scores_hbm, idx_hbm, out_hbm, in_v, out_v, idx_v,
                  tin_b, tout_b, isem0, isem1, osem0, osem1):
        wid = lax.axis_index("s") * NC + lax.axis_index("c")
        h = wid % 2
        r8 = pl.multiple_of((wid // 2) * ROWS, ROWS)
        neg_inf = jnp.full((L,), -jnp.inf, jnp.float32)
        isems = (isem0, isem1)
        osems = (osem0, osem1)

        def lo_of(k):
            return pl.multiple_of((2 * k + h) * CHW, CHW)

        pltpu.sync_copy(idx_hbm.at[pl.ds(r8, ROWS)], idx_v)

        def fill_row(rr, bi):
            def fill(i, carry):
                for u in range(8):
                    out_v[bi, rr, pl.ds((i * 8 + u) * L, L)] = neg_inf
                return carry
            lax.fori_loop(0, CHW // L // 8, fill, 0)

        def fill_rows(rr, carry):
            fill_row(rr, 0)
            fill_row(rr, 1)
            return carry

        lax.fori_loop(0, ROWS, fill_rows, 0)

        def issue_in(bi, lo):
            return pltpu.async_copy(
                scores_hbm.at[pl.ds(r8, ROWS), pl.ds(lo, CHW)],
                in_v.at[bi], isems[bi])

        def issue_out(bi, lo):
            return pltpu.async_copy(
                out_v.at[bi], out_hbm.at[pl.ds(r8, ROWS), pl.ds(lo, CHW)],
                osems[bi])

        def scan_rows(bi, lo, do_restore):
            def row_body(rr, carry):
                rowv = jnp.full((L,), 0, jnp.int32) + rr
                for j in range(K // L):
                    iv = idx_v[rr, pl.ds(j * L, L)]
                    m = (iv >= lo) & (iv < lo + CHW)
                    liv = iv - lo
                    if do_restore:
                        plsc.store_scatter(out_v.at[bi], [rowv, liv],
                                           neg_inf, mask=m)
                    else:
                        vv = plsc.load_gather(in_v.at[bi], [rowv, liv],
                                              mask=m)
                        plsc.store_scatter(out_v.at[bi], [rowv, liv], vv,
                                           mask=m)
                return carry
            lax.fori_loop(0, ROWS, row_body, 0)

        issue_in(0, lo_of(0))
        issue_in(1, lo_of(1))

        def chunk_pair(g, carry):
            for bi in range(2):
                k = 2 * g + bi
                lo = lo_of(k)

                @pl.when(g > 0)
                def _():
                    pltpu.make_async_copy(
                        out_v.at[bi],
                        out_hbm.at[pl.ds(r8, ROWS), pl.ds(lo, CHW)],
                        osems[bi]).wait()
                    scan_rows(bi, lo - 2 * 2 * CHW, do_restore=True)

                pltpu.make_async_copy(
                    scores_hbm.at[pl.ds(r8, ROWS), pl.ds(lo, CHW)],
                    in_v.at[bi], isems[bi]).wait()
                scan_rows(bi, lo, do_restore=False)

                @pl.when(g < NG - 1)
                def _():
                    issue_in(bi, pl.multiple_of(lo + 2 * 2 * CHW, CHW))

                issue_out(bi, lo)
            return carry

        lax.fori_loop(0, NG, chunk_pair, 0)

        last0, last1 = lo_of(NK - 2), lo_of(NK - 1)
        pltpu.make_async_copy(
            out_v.at[0], out_hbm.at[pl.ds(r8, ROWS), pl.ds(last0, CHW)],
            osems[0]).wait()
        pltpu.make_async_copy(
            out_v.at[1], out_hbm.at[pl.ds(r8, ROWS), pl.ds(last1, CHW)],
            osems[1]).wait()

        @pl.when(h == 1)
        def _():
            scan_rows(0, last0, do_restore=True)
            pltpu.sync_copy(
                scores_hbm.at[pl.ds(r8, ROWS), pl.ds(TAIL_LO, TAIL_A)],
                in_v.at[0].at[:, pl.ds(0, TAIL_A)])
            pltpu.sync_copy(
                scores_hbm.at[pl.ds(r8, ROWS), pl.ds(TB_LO, TAIL_B)], tin_b)

            def tb_fill(rr, carry):
                tout_b[rr, pl.ds(0, L)] = neg_inf
                tout_b[rr, pl.ds(L, L)] = neg_inf
                return carry
            lax.fori_loop(0, ROWS, tb_fill, 0)

            def row_body(rr, carry):
                rowv = jnp.full((L,), 0, jnp.int32) + rr
                for j in range(K // L):
                    iv = idx_v[rr, pl.ds(j * L, L)]
                    ma = (iv >= TAIL_LO) & (iv < TB_LO)
                    liv = iv - TAIL_LO
                    vv = plsc.load_gather(in_v.at[0], [rowv, liv], mask=ma)
                    plsc.store_scatter(out_v.at[0], [rowv, liv], vv, mask=ma)
                    mb = iv >= TB_LO
                    lbv = iv - TB_LO
                    vb = plsc.load_gather(tin_b, [rowv, lbv], mask=mb)
                    plsc.store_scatter(tout_b, [rowv, lbv], vb, mask=mb)
                return carry
            lax.fori_loop(0, ROWS, row_body, 0)
            pltpu.sync_copy(
                out_v.at[0].at[:, pl.ds(0, TAIL_A)],
                out_hbm.at[pl.ds(r8, ROWS), pl.ds(TAIL_LO, TAIL_A)])
            pltpu.sync_copy(
                tout_b, out_hbm.at[pl.ds(r8, ROWS), pl.ds(TB_LO, TAIL_B)])

    return sc_kernel(scores, allowed_idx)
